# baseline (speedup 1.0000x reference)
import functools

import jax
import jax.numpy as jnp
from jax import lax
from jax.experimental import pallas as pl
from jax.experimental.pallas import tpu as pltpu
from jax.experimental.pallas import tpu_sc as plsc

S = 2048
B = 4
D = 1024
ROW = B * D
NC = 2
NS = 16
NW = NC * NS
ROWS_PER_W = S // NW
CHUNK = 8
NBLK = ROWS_PER_W // CHUNK
LANES = 16


def _sc_pos_add(x2d, table):
    mesh = plsc.VectorSubcoreMesh(core_axis_name="c", subcore_axis_name="s")

    @functools.partial(
        pl.kernel,
        mesh=mesh,
        out_type=jax.ShapeDtypeStruct((S, ROW), jnp.float32),
        scratch_types=[
            pltpu.VMEM((CHUNK, ROW), jnp.float32),
            pltpu.VMEM((CHUNK, D), jnp.float32),
        ],
    )
    def k(x_hbm, t_hbm, out_hbm, xbuf, tbuf):
        wid = lax.axis_index("s") * NC + lax.axis_index("c")
        base = wid * ROWS_PER_W

        def blk_body(blk, carry):
            row0 = base + blk * CHUNK
            pltpu.sync_copy(x_hbm.at[pl.ds(row0, CHUNK)], xbuf)
            pltpu.sync_copy(t_hbm.at[pl.ds(row0, CHUNK)], tbuf)
            for i in range(CHUNK):
                def j_body(j, c, i=i):
                    off = j * LANES
                    t = tbuf[i, pl.ds(off, LANES)]
                    for b in range(B):
                        xbuf[i, pl.ds(b * D + off, LANES)] += t
                    return c
                lax.fori_loop(0, D // LANES, j_body, 0)
            pltpu.sync_copy(xbuf, out_hbm.at[pl.ds(row0, CHUNK)])
            return carry

        lax.fori_loop(0, NBLK, blk_body, 0)

    return k(x2d, table)


def kernel(x, pos_emb_table):
    x2d = x.reshape(S, ROW)
    out = _sc_pos_add(x2d, pos_emb_table)
    return out.reshape(S, B, D)


# speedup vs baseline: 1.0680x; 1.0680x over previous
import functools

import jax
import jax.numpy as jnp
from jax import lax
from jax.experimental import pallas as pl
from jax.experimental.pallas import tpu as pltpu
from jax.experimental.pallas import tpu_sc as plsc

S = 2048
B = 4
D = 1024
ROW = B * D
NC = 2
NS = 16
NW = NC * NS
ROWS_PER_W = S // NW
CHUNK = 8
NBLK = ROWS_PER_W // CHUNK
NBUF = 2
LANES = 16
JPR = D // LANES


def _sc_pos_add(xf, tf):
    mesh = plsc.VectorSubcoreMesh(core_axis_name="c", subcore_axis_name="s")

    @functools.partial(
        pl.kernel,
        mesh=mesh,
        out_type=jax.ShapeDtypeStruct((S * ROW,), jnp.float32),
        scratch_types=[
            pltpu.VMEM((NBUF, CHUNK * ROW), jnp.float32),
            pltpu.VMEM((NBUF, CHUNK * D), jnp.float32),
            pltpu.SemaphoreType.DMA,
            pltpu.SemaphoreType.DMA,
            pltpu.SemaphoreType.DMA,
            pltpu.SemaphoreType.DMA,
        ],
    )
    def k(x_hbm, t_hbm, out_hbm, xbuf, tbuf, l0, l1, s0, s1):
        wid = lax.axis_index("s") * NC + lax.axis_index("c")
        base = wid * ROWS_PER_W
        lsem = (l0, l1)
        ssem = (s0, s1)

        def start_load(blk, slot):
            r0 = base + blk * CHUNK
            pltpu.async_copy(
                x_hbm.at[pl.ds(r0 * ROW, CHUNK * ROW)], xbuf.at[slot], lsem[slot])
            pltpu.async_copy(
                t_hbm.at[pl.ds(r0 * D, CHUNK * D)], tbuf.at[slot], lsem[slot])

        def wait_load(slot):
            pltpu.make_async_copy(
                x_hbm.at[pl.ds(0, CHUNK * ROW)], xbuf.at[slot], lsem[slot]).wait()
            pltpu.make_async_copy(
                t_hbm.at[pl.ds(0, CHUNK * D)], tbuf.at[slot], lsem[slot]).wait()

        def start_store(blk, slot):
            r0 = base + blk * CHUNK
            pltpu.async_copy(
                xbuf.at[slot], out_hbm.at[pl.ds(r0 * ROW, CHUNK * ROW)], ssem[slot])

        def wait_store(slot):
            pltpu.make_async_copy(
                xbuf.at[slot], out_hbm.at[pl.ds(0, CHUNK * ROW)], ssem[slot]).wait()

        def compute(slot):
            xb = xbuf.at[slot]
            tb = tbuf.at[slot]

            @plsc.parallel_loop(0, CHUNK * JPR, unroll=4)
            def _(p):
                toff = p * LANES
                t = tb[pl.ds(toff, LANES)]
                xo = toff + (p // JPR) * (ROW - D)
                for b in range(B):
                    xb[pl.ds(xo + b * D, LANES)] += t

        for blk in range(NBLK):
            slot = blk % NBUF
            if blk == 0:
                start_load(0, 0)
            if blk + 1 < NBLK:
                nslot = (blk + 1) % NBUF
                if blk >= 1:
                    wait_store(nslot)
                start_load(blk + 1, nslot)
            wait_load(slot)
            compute(slot)
            start_store(blk, slot)
        wait_store((NBLK - 2) % NBUF)
        wait_store((NBLK - 1) % NBUF)

    return k(xf, tf)


def kernel(x, pos_emb_table):
    xf = x.reshape(S * ROW)
    tf = pos_emb_table.reshape(S * D)
    out = _sc_pos_add(xf, tf)
    return out.reshape(S, B, D)


# speedup vs baseline: 2.9504x; 2.7625x over previous
import functools

import jax
import jax.numpy as jnp
from jax import lax
from jax.experimental import pallas as pl
from jax.experimental.pallas import tpu as pltpu
from jax.experimental.pallas import tpu_sc as plsc

S = 2048
B = 4
D = 1024
NC = 2
NS = 16
NW = NC * NS
ROWS_PER_W = S // NW
CHUNK = 8
NBLK = ROWS_PER_W // CHUNK
NBUF = 2
LANES = 16
JPR = D // LANES


def _sc_pos_add(x, table):
    mesh = plsc.VectorSubcoreMesh(core_axis_name="c", subcore_axis_name="s")

    @functools.partial(
        pl.kernel,
        mesh=mesh,
        out_type=jax.ShapeDtypeStruct((S, B, D), jnp.float32),
        scratch_types=[
            pltpu.VMEM((NBUF, CHUNK, B, D), jnp.float32),
            pltpu.VMEM((NBUF, CHUNK, D), jnp.float32),
            pltpu.SemaphoreType.DMA,
            pltpu.SemaphoreType.DMA,
            pltpu.SemaphoreType.DMA,
            pltpu.SemaphoreType.DMA,
        ],
    )
    def k(x_hbm, t_hbm, out_hbm, xbuf, tbuf, l0, l1, s0, s1):
        wid = lax.axis_index("s") * NC + lax.axis_index("c")
        base = wid * ROWS_PER_W
        lsem = (l0, l1)
        ssem = (s0, s1)

        def start_load(blk, slot):
            r0 = base + blk * CHUNK
            pltpu.async_copy(
                x_hbm.at[pl.ds(r0, CHUNK)], xbuf.at[slot], lsem[slot])
            pltpu.async_copy(
                t_hbm.at[pl.ds(r0, CHUNK)], tbuf.at[slot], lsem[slot])

        def wait_load(slot):
            pltpu.make_async_copy(
                x_hbm.at[pl.ds(0, CHUNK)], xbuf.at[slot], lsem[slot]).wait()
            pltpu.make_async_copy(
                t_hbm.at[pl.ds(0, CHUNK)], tbuf.at[slot], lsem[slot]).wait()

        def start_store(blk, slot):
            r0 = base + blk * CHUNK
            pltpu.async_copy(
                xbuf.at[slot], out_hbm.at[pl.ds(r0, CHUNK)], ssem[slot])

        def wait_store(slot):
            pltpu.make_async_copy(
                xbuf.at[slot], out_hbm.at[pl.ds(0, CHUNK)], ssem[slot]).wait()

        def compute(slot):
            xb = xbuf.at[slot]
            tb = tbuf.at[slot]

            @pl.loop(0, CHUNK)
            def _(i):
                @plsc.parallel_loop(0, JPR, unroll=4)
                def _(j):
                    jo = j * LANES
                    t = tb[i, pl.ds(jo, LANES)]
                    for b in range(B):
                        xb[i, b, pl.ds(jo, LANES)] += t

        for blk in range(NBLK):
            slot = blk % NBUF
            if blk == 0:
                start_load(0, 0)
            if blk + 1 < NBLK:
                nslot = (blk + 1) % NBUF
                if blk >= 1:
                    wait_store(nslot)
                start_load(blk + 1, nslot)
            wait_load(slot)
            compute(slot)
            start_store(blk, slot)
        wait_store((NBLK - 2) % NBUF)
        wait_store((NBLK - 1) % NBUF)

    return k(x, table)


def kernel(x, pos_emb_table):
    return _sc_pos_add(x, pos_emb_table)
